# baseline (speedup 1.0000x reference)
import functools

import jax
import jax.numpy as jnp
from jax import lax
from jax.experimental import pallas as pl
from jax.experimental.pallas import tpu as pltpu
from jax.experimental.pallas import tpu_sc as plsc


def _sc_gather(idx, table):
    B = idx.shape[0]
    D = table.shape[1]
    info = plsc.get_sparse_core_info()
    nw = info.num_cores * info.num_subcores
    b_per_w = B // nw

    mesh = plsc.VectorSubcoreMesh(core_axis_name="c", subcore_axis_name="s")

    @functools.partial(
        pl.kernel,
        mesh=mesh,
        out_type=jax.ShapeDtypeStruct((B, D), jnp.float32),
        scratch_types=[
            pltpu.VMEM((b_per_w,), jnp.int32),
            pltpu.VMEM((b_per_w, D), jnp.float32),
            pltpu.SemaphoreType.DMA,
        ],
        compiler_params=pltpu.CompilerParams(use_tc_tiling_on_sc=False),
    )
    def gather_k(idx_hbm, table_hbm, out_hbm, idx_v, rows_v, sem):
        wid = lax.axis_index("s") * info.num_cores + lax.axis_index("c")
        base = wid * b_per_w
        pltpu.sync_copy(idx_hbm.at[pl.ds(base, b_per_w)], idx_v)
        pltpu.async_copy(table_hbm.at[idx_v], rows_v, sem).wait()
        pltpu.sync_copy(rows_v, out_hbm.at[pl.ds(base, b_per_w)])

    return gather_k(idx, table)


def _mm_body(e_ref, w_ref, b_ref, o_ref):
    e = e_ref[...]
    norm = jnp.sqrt(jnp.sum(e * e, axis=1, keepdims=True))
    scale = jnp.minimum(1.0, 1.0 / jnp.maximum(norm, 1e-12))
    en = e * scale
    o_ref[...] = (
        jnp.dot(en, w_ref[...], preferred_element_type=jnp.float32) + b_ref[...]
    )


def _tc_project(e, w, b2, block_m, block_n):
    B, D = e.shape
    V = w.shape[1]
    grid = (pl.cdiv(V, block_n), pl.cdiv(B, block_m))
    return pl.pallas_call(
        _mm_body,
        grid=grid,
        in_specs=[
            pl.BlockSpec((block_m, D), lambda nc, nr: (nr, 0)),
            pl.BlockSpec((D, block_n), lambda nc, nr: (0, nc)),
            pl.BlockSpec((1, block_n), lambda nc, nr: (0, nc)),
        ],
        out_specs=pl.BlockSpec((block_m, block_n), lambda nc, nr: (nr, nc)),
        out_shape=jax.ShapeDtypeStruct((B, V), jnp.float32),
        compiler_params=pltpu.CompilerParams(
            dimension_semantics=("parallel", "parallel"),
        ),
    )(e, w, b2)


def kernel(inputs_, emb_table, W, b):
    e = emb_table[:1024]
    return _tc_project(e, W, b.reshape(1, -1), block_m=32, block_n=100000)


# speedup vs baseline: 2.2529x; 2.2529x over previous
import functools

import jax
import jax.numpy as jnp
from jax import lax
from jax.experimental import pallas as pl
from jax.experimental.pallas import tpu as pltpu
from jax.experimental.pallas import tpu_sc as plsc


def _sc_gather(idx, table):
    B = idx.shape[0]
    D = table.shape[1]
    info = plsc.get_sparse_core_info()
    nw = info.num_cores * info.num_subcores
    b_per_w = B // nw

    mesh = plsc.VectorSubcoreMesh(core_axis_name="c", subcore_axis_name="s")

    @functools.partial(
        pl.kernel,
        mesh=mesh,
        out_type=jax.ShapeDtypeStruct((B, D), jnp.float32),
        scratch_types=[
            pltpu.VMEM((b_per_w,), jnp.int32),
            pltpu.VMEM((b_per_w, D), jnp.float32),
            pltpu.SemaphoreType.DMA,
        ],
        compiler_params=pltpu.CompilerParams(use_tc_tiling_on_sc=False),
    )
    def gather_k(idx_hbm, table_hbm, out_hbm, idx_v, rows_v, sem):
        wid = lax.axis_index("s") * info.num_cores + lax.axis_index("c")
        base = wid * b_per_w
        pltpu.sync_copy(idx_hbm.at[pl.ds(base, b_per_w)], idx_v)
        pltpu.async_copy(table_hbm.at[idx_v], rows_v, sem).wait()
        pltpu.sync_copy(rows_v, out_hbm.at[pl.ds(base, b_per_w)])

    return gather_k(idx, table)


def _mmt_body(e_ref, w_ref, b_ref, o_ref, ent_ref):
    @pl.when(pl.program_id(0) == 0)
    def _():
        e = e_ref[...]
        norm = jnp.sqrt(jnp.sum(e * e, axis=1, keepdims=True))
        scale = jnp.minimum(1.0, 1.0 / jnp.maximum(norm, 1e-12))
        en = e * scale
        ent_ref[0:64, :] = en.T
        ent_ref[64:65, :] = jnp.ones((1, en.shape[0]), jnp.float32)

    w_aug = jnp.concatenate([w_ref[...], b_ref[...]], axis=0)
    o_ref[...] = lax.dot_general(
        w_aug,
        ent_ref[...],
        dimension_numbers=(((0,), (0,)), ((), ())),
        preferred_element_type=jnp.float32,
    )


def _tc_project_t(e, w, b2, block_v):
    B, D = e.shape
    V = w.shape[1]
    grid = (pl.cdiv(V, block_v),)
    return pl.pallas_call(
        _mmt_body,
        grid=grid,
        in_specs=[
            pl.BlockSpec((B, D), lambda j: (0, 0)),
            pl.BlockSpec((D, block_v), lambda j: (0, j)),
            pl.BlockSpec((1, block_v), lambda j: (0, j)),
        ],
        out_specs=pl.BlockSpec((block_v, B), lambda j: (j, 0)),
        out_shape=jax.ShapeDtypeStruct((V, B), jnp.float32),
        scratch_shapes=[pltpu.VMEM((D + 1, B), jnp.float32)],
        compiler_params=pltpu.CompilerParams(
            dimension_semantics=("arbitrary",),
        ),
    )(e, w, b2)


def kernel(inputs_, emb_table, W, b):
    idx = inputs_.astype(jnp.int32)
    e = _sc_gather(idx, emb_table)
    out_t = _tc_project_t(e, W, b.reshape(1, -1), block_v=2048)
    return out_t.T


# speedup vs baseline: 2.5766x; 1.1437x over previous
import functools

import jax
import jax.numpy as jnp
from jax import lax
from jax.experimental import pallas as pl
from jax.experimental.pallas import tpu as pltpu
from jax.experimental.pallas import tpu_sc as plsc


def _sc_gather(idx, table):
    B = idx.shape[0]
    D = table.shape[1]
    info = plsc.get_sparse_core_info()
    nw = info.num_cores * info.num_subcores
    nl = info.num_lanes
    b_per_w = B // nw

    mesh = plsc.VectorSubcoreMesh(core_axis_name="c", subcore_axis_name="s")

    @functools.partial(
        pl.kernel,
        mesh=mesh,
        out_type=jax.ShapeDtypeStruct((B, D), jnp.float32),
        scratch_types=[
            pltpu.VMEM((b_per_w,), jnp.int32),
            pltpu.VMEM((b_per_w, D), jnp.float32),
            pltpu.SemaphoreType.DMA,
        ],
        compiler_params=pltpu.CompilerParams(needs_layout_passes=False),
    )
    def gather_k(idx_hbm, table_hbm, out_hbm, idx_v, rows_v, sem):
        wid = lax.axis_index("s") * info.num_cores + lax.axis_index("c")
        base = wid * b_per_w
        pltpu.sync_copy(idx_hbm.at[pl.ds(base, b_per_w)], idx_v)
        lane = lax.iota(jnp.int32, nl)
        copies = []
        for i in range(b_per_w):
            chunk = idx_v[pl.ds((i // nl) * nl, nl)]
            sel = jnp.where(lane == (i % nl), chunk, 0)
            row = lax.reduce_sum_p.bind(sel, axes=(0,))
            copies.append(
                pltpu.make_async_copy(
                    table_hbm.at[pl.ds(row, 1)], rows_v.at[pl.ds(i, 1)], sem
                )
            )
            copies[-1].start()
        for c in copies:
            c.wait()
        pltpu.sync_copy(rows_v, out_hbm.at[pl.ds(base, b_per_w)])

    return gather_k(idx, table)


def _mmt_body(e_ref, w_ref, b_ref, o_ref, ent_ref):
    @pl.when(pl.program_id(0) == 0)
    def _():
        e = e_ref[...]
        norm = jnp.sqrt(jnp.sum(e * e, axis=1, keepdims=True))
        scale = jnp.minimum(1.0, 1.0 / jnp.maximum(norm, 1e-12))
        en = e * scale
        ent_ref[0:64, :] = en.T
        ent_ref[64:65, :] = jnp.ones((1, en.shape[0]), jnp.float32)

    w_aug = jnp.concatenate([w_ref[...], b_ref[...]], axis=0)
    o_ref[...] = lax.dot_general(
        w_aug,
        ent_ref[...],
        dimension_numbers=(((0,), (0,)), ((), ())),
        preferred_element_type=jnp.float32,
    )


def _tc_project_t(e, w, b2, block_v):
    B, D = e.shape
    V = w.shape[1]
    grid = (pl.cdiv(V, block_v),)
    return pl.pallas_call(
        _mmt_body,
        grid=grid,
        in_specs=[
            pl.BlockSpec((B, D), lambda j: (0, 0)),
            pl.BlockSpec((D, block_v), lambda j: (0, j)),
            pl.BlockSpec((1, block_v), lambda j: (0, j)),
        ],
        out_specs=pl.BlockSpec((block_v, B), lambda j: (j, 0)),
        out_shape=jax.ShapeDtypeStruct((V, B), jnp.float32),
        scratch_shapes=[pltpu.VMEM((D + 1, B), jnp.float32)],
        compiler_params=pltpu.CompilerParams(
            dimension_semantics=("arbitrary",),
        ),
    )(e, w, b2)


def kernel(inputs_, emb_table, W, b):
    idx = inputs_.astype(jnp.int32)
    e = _sc_gather(idx, emb_table)
    out_t = _tc_project_t(e, W, b.reshape(1, -1), block_v=2048)
    return out_t.T
